# Initial kernel scaffold; baseline (speedup 1.0000x reference)
import jax
import jax.numpy as jnp
from jax.experimental import pallas as pl


def kernel(x, target):
    raise NotImplementedError("write your pallas kernel here")


# speedup vs baseline: 5.3669x; 5.3669x over previous
import math

import jax
import jax.numpy as jnp
from jax import lax
from jax.experimental import pallas as pl
from jax.experimental.pallas import tpu as pltpu

_SIZE = 32000
_SMOOTHING = 0.1
_CONF = 1.0 - _SMOOTHING
_FILL = _SMOOTHING / (_SIZE - 2)
_ROW_K = _CONF * math.log(_CONF) + (_SIZE - 2) * _FILL * math.log(_FILL)

_N = 2048
_BN = 128
_BV = 6400


def _body(t_ref, x_ref, out_ref):
    i = pl.program_id(0)
    j = pl.program_id(1)
    t = t_ref[0, 0, :]
    x = x_ref[...]
    col = lax.broadcasted_iota(jnp.int32, (_BN, _BV), 1) + j * _BV
    tcol = t[:, None]
    w = jnp.where(col == tcol, _CONF, _FILL)
    w = jnp.where(col == 0, 0.0, w)
    m = (tcol != 0).astype(x.dtype)
    partial = jnp.sum(x * (w * m))

    @pl.when((i == 0) & (j == 0))
    def _init():
        out_ref[0, 0] = 0.0

    @pl.when(j == 0)
    def _const_term():
        out_ref[0, 0] += jnp.sum(m) * _ROW_K

    out_ref[0, 0] += -partial


def kernel(x, target):
    n, size = x.shape
    assert (n, size) == (_N, _SIZE)
    t3 = target.astype(jnp.int32).reshape(_N // _BN, 1, _BN)
    grid = (_N // _BN, _SIZE // _BV)
    out = pl.pallas_call(
        _body,
        grid=grid,
        in_specs=[
            pl.BlockSpec((1, 1, _BN), lambda i, j: (i, 0, 0)),
            pl.BlockSpec((_BN, _BV), lambda i, j: (i, j)),
        ],
        out_specs=pl.BlockSpec(
            (1, 1), lambda i, j: (0, 0), memory_space=pltpu.SMEM
        ),
        out_shape=jax.ShapeDtypeStruct((1, 1), jnp.float32),
        compiler_params=pltpu.CompilerParams(
            dimension_semantics=("arbitrary", "arbitrary"),
        ),
    )(t3, x)
    return out[0, 0]
